# baseline (speedup 1.0000x reference)
import functools

import jax
import jax.numpy as jnp
from jax import lax
from jax.experimental import pallas as pl
from jax.experimental.pallas import tpu as pltpu
import jax.experimental.pallas.tpu_sc as plsc

N = 360
E = 11520
B = 512
BB = 16
BBC = 64
ROWS_PER_TILE = 12
NPAD = 368
F32 = jnp.float32


def _sc_body(ei_hbm, w_hbm, out_hbm,
             src_v, dst_v, w_v, deg16, dinv_v, slab16, slab_v):
    c = lax.axis_index("c")
    s = lax.axis_index("s")
    wid = s * 2 + c
    lo = wid * ROWS_PER_TILE

    pltpu.sync_copy(ei_hbm.at[0], src_v)
    pltpu.sync_copy(ei_hbm.at[1], dst_v)
    pltpu.sync_copy(w_hbm, w_v)

    lanes = lax.iota(jnp.int32, 16)
    zeros16 = jnp.zeros((16,), F32)

    def zero_deg(i, _):
        for u in range(8):
            deg16[pl.ds((i * 8 + u) * 16, 16)] = zeros16
        return 0
    lax.fori_loop(0, (16 * NPAD) // (16 * 8), zero_deg, 0)

    def zero_slab(i, _):
        for u in range(8):
            slab16[pl.ds((i * 8 + u) * 16, 16)] = zeros16
        return 0
    lax.fori_loop(0, (16 * ROWS_PER_TILE * N) // (16 * 8), zero_slab, 0)

    lane_deg = lanes * NPAD
    def deg_step(e, _):
        for u in range(4):
            sv = src_v[pl.ds((e * 4 + u) * 16, 16)]
            wv = w_v[pl.ds((e * 4 + u) * 16, 16)]
            plsc.addupdate_scatter(deg16, [lane_deg + sv], wv)
        return 0
    lax.fori_loop(0, E // 64, deg_step, 0)

    def dinv_step(k, _):
        acc = deg16[pl.ds(k * 16, 16)]
        for l in range(1, 16):
            acc = acc + deg16[pl.ds(l * NPAD + k * 16, 16)]
        bits = plsc.bitcast(acc, jnp.int32)
        y = plsc.bitcast(jnp.int32(0x5F3759DF) - (bits >> 1), F32)
        for _i in range(4):
            y = y * (1.5 - 0.5 * acc * y * y)
        dinv_v[pl.ds(k * 16, 16)] = jnp.where(acc > 0.0, y, 0.0)
        return 0
    lax.fori_loop(0, NPAD // 16, dinv_step, 0)

    lane_slab = lanes * (ROWS_PER_TILE * N)
    lo_v = jnp.full((16,), lo, jnp.int32)
    def edge_step(e, _):
        for u in range(4):
            sv = src_v[pl.ds((e * 4 + u) * 16, 16)]
            dv = dst_v[pl.ds((e * 4 + u) * 16, 16)]
            wv = w_v[pl.ds((e * 4 + u) * 16, 16)]
            rel = dv - lo_v
            m = (rel >= 0) & (rel < ROWS_PER_TILE)
            ds_ = plsc.load_gather(dinv_v, [sv])
            dd = plsc.load_gather(dinv_v, [dv])
            norm = -(ds_ * wv * dd)
            addr = lane_slab + rel * N + sv
            addr = jnp.where(m, addr, 0)
            plsc.addupdate_scatter(slab16, [addr], norm, mask=m)
        return 0
    lax.fori_loop(0, E // 64, edge_step, 0)

    def slab_step(k, _):
        acc = slab16[pl.ds(k * 16, 16)]
        for l in range(1, 16):
            acc = acc + slab16[pl.ds(l * (ROWS_PER_TILE * N) + k * 16, 16)]
        slab_v[pl.ds(k * 16, 16)] = acc
        return 0
    lax.fori_loop(0, (ROWS_PER_TILE * N) // 16, slab_step, 0)

    @pl.when(wid < N // ROWS_PER_TILE)
    def _():
        pltpu.sync_copy(slab_v, out_hbm.at[pl.ds(lo * N, ROWS_PER_TILE * N)])


def _build_lhat(ei, w):
    mesh = plsc.VectorSubcoreMesh(core_axis_name="c", subcore_axis_name="s")
    f = pl.kernel(
        _sc_body,
        out_type=jax.ShapeDtypeStruct((N * N,), F32),
        mesh=mesh,
        scratch_types=[
            pltpu.VMEM((E,), jnp.int32),
            pltpu.VMEM((E,), jnp.int32),
            pltpu.VMEM((E,), F32),
            pltpu.VMEM((16 * NPAD,), F32),
            pltpu.VMEM((NPAD,), F32),
            pltpu.VMEM((16 * ROWS_PER_TILE * N,), F32),
            pltpu.VMEM((ROWS_PER_TILE * N,), F32),
        ],
        compiler_params=pltpu.CompilerParams(needs_layout_passes=False),
        name="sc_build_lhat",
    )
    return f(ei, w).reshape(N, N)


def _st_core(x_ref, x_t, L, Wt1s, bt1s, KCe, cbe, Wt2e, bt2e, t_in):
    tm = t_in - 2
    to = t_in - 4
    xs = [x_t(t) for t in range(t_in)]
    h1 = []
    for t in range(tm):
        xwin = jnp.concatenate([xs[t], xs[t + 1], xs[t + 2]], axis=0)
        g = jnp.dot(Wt1s, xwin, preferred_element_type=F32) + bt1s
        h1.append(jnp.maximum(
            g[0:12 * BB] * jax.nn.sigmoid(g[12 * BB:24 * BB])
            + g[24 * BB:36 * BB], 0.0))
    h1_all = jnp.concatenate(h1, axis=0)
    b_all = lax.dot_general(h1_all, L, (((1,), (1,)), ((), ())),
                            preferred_element_type=F32)
    c1 = []
    cw = 12 * BB
    for t in range(tm):
        hb = jnp.concatenate([h1[t], b_all[t * cw:(t + 1) * cw]], axis=0)
        c1.append(jnp.maximum(
            jnp.dot(KCe, hb, preferred_element_type=F32) + cbe, 0.0))
    h2 = []
    for t in range(to):
        cwin = jnp.concatenate([c1[t], c1[t + 1], c1[t + 2]], axis=0)
        g2 = jnp.dot(Wt2e, cwin, preferred_element_type=F32) + bt2e
        h2.append(jnp.maximum(
            g2[0:BB] * jax.nn.sigmoid(g2[BB:2 * BB]) + g2[2 * BB:3 * BB],
            0.0))
    return h2


def _write_out(h2, t2_ref, stats_ref):
    s1 = None
    s2 = None
    for t, h in enumerate(h2):
        t2_ref[t] = h
        a = jnp.sum(h, axis=0)
        b = jnp.sum(h * h, axis=0)
        s1 = a if s1 is None else s1 + a
        s2 = b if s2 is None else s2 + b
    stats_ref[...] += jnp.stack([s1, s2])


def _bn_scale_shift(sin_ref, gam, bet, cnt):
    mean = sin_ref[0:1, :] / cnt
    var = sin_ref[1:2, :] / cnt - mean * mean
    scale = gam[...] * lax.rsqrt(var + 1e-5)
    shift = bet[...] - mean * scale
    return scale, shift


def _body_r1(x_ref, L_ref, Wt1e, bt1e, KCe, cbe, Wt2e, bt2e,
             t2_ref, stats_ref):
    @pl.when(pl.program_id(0) == 0)
    def _():
        stats_ref[...] = jnp.zeros_like(stats_ref)
    h2 = _st_core(x_ref, lambda t: x_ref[t], L_ref[...], Wt1e[...],
                  bt1e[...], KCe[...], cbe[...], Wt2e[...], bt2e[...], 12)
    _write_out(h2, t2_ref, stats_ref)


def _body_r2(x_ref, sin_ref, gam, bet, L_ref, Wt1e, bt1e, KCe, cbe, Wt2e,
             bt2e, t2_ref, stats_ref):
    @pl.when(pl.program_id(0) == 0)
    def _():
        stats_ref[...] = jnp.zeros_like(stats_ref)
    scale, shift = _bn_scale_shift(sin_ref, gam, bet, float(B * 8))

    def x_t(t):
        return jnp.maximum(x_ref[t] * scale + shift, 0.0)

    h2 = _st_core(x_ref, x_t, L_ref[...], Wt1e[...], bt1e[...], KCe[...],
                  cbe[...], Wt2e[...], bt2e[...], 8)
    _write_out(h2, t2_ref, stats_ref)


def _body_out(x_ref, sin_ref, gam, bet, lw_ref, lb_ref, out_ref):
    scale, shift = _bn_scale_shift(sin_ref, gam, bet, float(B * 4))
    h = jnp.maximum(x_ref[3] * scale + shift, 0.0)
    out = lax.dot_general(h, lw_ref[...], (((1,), (1,)), ((), ())),
                          preferred_element_type=F32)
    out_ref[...] = out + lb_ref[...]


def _full(shape):
    if len(shape) == 2:
        return pl.BlockSpec(shape, lambda i: (0, 0))
    return pl.BlockSpec(shape, lambda i: (0, 0, 0))


def _prep_block(p, pref):
    eye = jnp.eye(BB, dtype=F32)
    wt1e = jnp.kron(jnp.concatenate(
        [p[pref + '_t1_w%d' % j].reshape(12, 3) for j in (1, 2, 3)],
        axis=0), eye)
    bt1e = jnp.repeat(jnp.concatenate(
        [p[pref + '_t1_b%d' % j] for j in (1, 2, 3)]), BB).reshape(36 * BB, 1)
    kce = jnp.concatenate([jnp.kron(p[pref + '_cw0'], eye),
                           jnp.kron(p[pref + '_cw1'], eye)], axis=1)
    cbe = jnp.repeat(p[pref + '_cb'], BB).reshape(12 * BB, 1)
    w2s = [p[pref + '_t2_w%d' % j].reshape(12, 3) for j in (1, 2, 3)]
    w2flat = jnp.stack([w.T.reshape(36) for w in w2s], axis=0)
    wt2e = jnp.kron(w2flat, eye)
    bt2e = jnp.repeat(jnp.stack(
        [p[pref + '_t2_b%d' % j][0] for j in (1, 2, 3)]), BB).reshape(3 * BB, 1)
    return (wt1e, bt1e, kce, cbe, wt2e, bt2e)


def _param_specs():
    return [_full((36 * BB, 3 * BB)), _full((36 * BB, 1)),
            _full((12 * BB, 24 * BB)), _full((12 * BB, 1)),
            _full((3 * BB, 36 * BB)), _full((3 * BB, 1))]


def _dense_forward(L, x3, params):
    pr1 = _prep_block(params, 'r1')
    pr2 = _prep_block(params, 'r2')
    grid = (B // BB,)

    t2a, stats_a = pl.pallas_call(
        _body_r1,
        grid=grid,
        in_specs=[pl.BlockSpec((12, BB, N), lambda i: (0, i, 0)),
                  _full((N, N))] + _param_specs(),
        out_specs=[pl.BlockSpec((8, BB, N), lambda i: (0, i, 0)),
                   _full((2, N))],
        out_shape=[jax.ShapeDtypeStruct((8, B, N), F32),
                   jax.ShapeDtypeStruct((2, N), F32)],
    )(x3, L, *pr1)

    t2b, stats_b = pl.pallas_call(
        _body_r2,
        grid=grid,
        in_specs=[pl.BlockSpec((8, BB, N), lambda i: (0, i, 0)),
                  _full((2, N)), _full((1, N)), _full((1, N)),
                  _full((N, N))] + _param_specs(),
        out_specs=[pl.BlockSpec((4, BB, N), lambda i: (0, i, 0)),
                   _full((2, N))],
        out_shape=[jax.ShapeDtypeStruct((4, B, N), F32),
                   jax.ShapeDtypeStruct((2, N), F32)],
    )(t2a, stats_a, params['r1_bn_g'].reshape(1, N),
      params['r1_bn_b'].reshape(1, N), L, *pr2)

    out = pl.pallas_call(
        _body_out,
        grid=(B // BBC,),
        in_specs=[pl.BlockSpec((4, BBC, N), lambda i: (0, i, 0)),
                  _full((2, N)), _full((1, N)), _full((1, N)),
                  _full((N, N)), _full((1, N))],
        out_specs=pl.BlockSpec((BBC, N), lambda i: (i, 0)),
        out_shape=jax.ShapeDtypeStruct((B, N), F32),
    )(t2b, stats_b, params['r2_bn_g'].reshape(1, N),
      params['r2_bn_b'].reshape(1, N), params['lin_w'],
      params['lin_b'].reshape(1, N))
    return out


def kernel(x, edge_weight, params, edge_index):
    L = _build_lhat(edge_index.astype(jnp.int32), edge_weight.astype(F32))
    x3 = jnp.transpose(x[..., 0].astype(F32), (1, 0, 2))
    return _dense_forward(L, x3, params)


# speedup vs baseline: 1.0177x; 1.0177x over previous
import jax
import jax.numpy as jnp
from jax import lax
from jax.experimental import pallas as pl
from jax.experimental.pallas import tpu as pltpu
import jax.experimental.pallas.tpu_sc as plsc

N = 360
E = 11520
B = 512
BB = 8
BBC = 64
ROWS_PER_TILE = 12
NPAD = 368
F32 = jnp.float32


def _sc_body(ei_hbm, w_hbm, out_hbm,
             src_v, dst_v, w_v, deg16, dinv_v, slab16, slab_v):
    c = lax.axis_index("c")
    s = lax.axis_index("s")
    wid = s * 2 + c
    lo = wid * ROWS_PER_TILE

    pltpu.sync_copy(ei_hbm.at[0], src_v)
    pltpu.sync_copy(ei_hbm.at[1], dst_v)
    pltpu.sync_copy(w_hbm, w_v)

    lanes = lax.iota(jnp.int32, 16)
    zeros16 = jnp.zeros((16,), F32)

    def zero_deg(i, _):
        for u in range(8):
            deg16[pl.ds((i * 8 + u) * 16, 16)] = zeros16
        return 0
    lax.fori_loop(0, (16 * NPAD) // (16 * 8), zero_deg, 0)

    def zero_slab(i, _):
        for u in range(8):
            slab16[pl.ds((i * 8 + u) * 16, 16)] = zeros16
        return 0
    lax.fori_loop(0, (16 * ROWS_PER_TILE * N) // (16 * 8), zero_slab, 0)

    lane_deg = lanes * NPAD
    def deg_step(e, _):
        for u in range(4):
            sv = src_v[pl.ds((e * 4 + u) * 16, 16)]
            wv = w_v[pl.ds((e * 4 + u) * 16, 16)]
            plsc.addupdate_scatter(deg16, [lane_deg + sv], wv)
        return 0
    lax.fori_loop(0, E // 64, deg_step, 0)

    def dinv_step(k, _):
        acc = deg16[pl.ds(k * 16, 16)]
        for l in range(1, 16):
            acc = acc + deg16[pl.ds(l * NPAD + k * 16, 16)]
        bits = plsc.bitcast(acc, jnp.int32)
        y = plsc.bitcast(jnp.int32(0x5F3759DF) - (bits >> 1), F32)
        for _i in range(4):
            y = y * (1.5 - 0.5 * acc * y * y)
        dinv_v[pl.ds(k * 16, 16)] = jnp.where(acc > 0.0, y, 0.0)
        return 0
    lax.fori_loop(0, NPAD // 16, dinv_step, 0)

    lane_slab = lanes * (ROWS_PER_TILE * N)
    lo_v = jnp.full((16,), lo, jnp.int32)
    def edge_step(e, _):
        for u in range(4):
            sv = src_v[pl.ds((e * 4 + u) * 16, 16)]
            dv = dst_v[pl.ds((e * 4 + u) * 16, 16)]
            wv = w_v[pl.ds((e * 4 + u) * 16, 16)]
            rel = dv - lo_v
            m = (rel >= 0) & (rel < ROWS_PER_TILE)
            ds_ = plsc.load_gather(dinv_v, [sv])
            dd = plsc.load_gather(dinv_v, [dv])
            norm = -(ds_ * wv * dd)
            addr = lane_slab + rel * N + sv
            addr = jnp.where(m, addr, 0)
            plsc.addupdate_scatter(slab16, [addr], norm, mask=m)
        return 0
    lax.fori_loop(0, E // 64, edge_step, 0)

    def slab_step(k, _):
        acc = slab16[pl.ds(k * 16, 16)]
        for l in range(1, 16):
            acc = acc + slab16[pl.ds(l * (ROWS_PER_TILE * N) + k * 16, 16)]
        slab_v[pl.ds(k * 16, 16)] = acc
        return 0
    lax.fori_loop(0, (ROWS_PER_TILE * N) // 16, slab_step, 0)

    @pl.when(wid < N // ROWS_PER_TILE)
    def _():
        pltpu.sync_copy(slab_v, out_hbm.at[pl.ds(lo * N, ROWS_PER_TILE * N)])


def _build_lhat(ei, w):
    mesh = plsc.VectorSubcoreMesh(core_axis_name="c", subcore_axis_name="s")
    f = pl.kernel(
        _sc_body,
        out_type=jax.ShapeDtypeStruct((N * N,), F32),
        mesh=mesh,
        scratch_types=[
            pltpu.VMEM((E,), jnp.int32),
            pltpu.VMEM((E,), jnp.int32),
            pltpu.VMEM((E,), F32),
            pltpu.VMEM((16 * NPAD,), F32),
            pltpu.VMEM((NPAD,), F32),
            pltpu.VMEM((16 * ROWS_PER_TILE * N,), F32),
            pltpu.VMEM((ROWS_PER_TILE * N,), F32),
        ],
        compiler_params=pltpu.CompilerParams(needs_layout_passes=False),
        name="sc_build_lhat",
    )
    return f(ei, w).reshape(N, N)


def _st_core(x_ref, x_t, L, Wt1s, bt1s, KCe, cbe, Wt2e, bt2e, t_in):
    tm = t_in - 2
    to = t_in - 4
    xs = [x_t(t) for t in range(t_in)]
    h1 = []
    for t in range(tm):
        xwin = jnp.concatenate([xs[t], xs[t + 1], xs[t + 2]], axis=0)
        g = jnp.dot(Wt1s, xwin, preferred_element_type=F32) + bt1s
        h1.append(jnp.maximum(
            g[0:12 * BB] * jax.nn.sigmoid(g[12 * BB:24 * BB])
            + g[24 * BB:36 * BB], 0.0))
    h1_all = jnp.concatenate(h1, axis=0)
    b_all = lax.dot_general(h1_all, L, (((1,), (1,)), ((), ())),
                            preferred_element_type=F32)
    c1 = []
    cw = 12 * BB
    for t in range(tm):
        hb = jnp.concatenate([h1[t], b_all[t * cw:(t + 1) * cw]], axis=0)
        c1.append(jnp.maximum(
            jnp.dot(KCe, hb, preferred_element_type=F32) + cbe, 0.0))
    h2 = []
    for t in range(to):
        cwin = jnp.concatenate([c1[t], c1[t + 1], c1[t + 2]], axis=0)
        g2 = jnp.dot(Wt2e, cwin, preferred_element_type=F32) + bt2e
        h2.append(jnp.maximum(
            g2[0:BB] * jax.nn.sigmoid(g2[BB:2 * BB]) + g2[2 * BB:3 * BB],
            0.0))
    return h2


def _write_out(h2, t2_ref, stats_ref):
    s1 = None
    s2 = None
    for t, h in enumerate(h2):
        t2_ref[t] = h
        a = jnp.sum(h, axis=0)
        b = jnp.sum(h * h, axis=0)
        s1 = a if s1 is None else s1 + a
        s2 = b if s2 is None else s2 + b
    stats_ref[...] += jnp.stack([s1, s2])


def _bn_scale_shift(sin_ref, gam, bet, cnt):
    mean = sin_ref[0:1, :] / cnt
    var = sin_ref[1:2, :] / cnt - mean * mean
    scale = gam[...] * lax.rsqrt(var + 1e-5)
    shift = bet[...] - mean * scale
    return scale, shift


def _body_r1(x_ref, L_ref, Wt1e, bt1e, KCe, cbe, Wt2e, bt2e,
             t2_ref, stats_ref):
    @pl.when(pl.program_id(0) == 0)
    def _():
        stats_ref[...] = jnp.zeros_like(stats_ref)
    h2 = _st_core(x_ref, lambda t: x_ref[t], L_ref[...], Wt1e[...],
                  bt1e[...], KCe[...], cbe[...], Wt2e[...], bt2e[...], 12)
    _write_out(h2, t2_ref, stats_ref)


def _body_r2(x_ref, sin_ref, gam, bet, L_ref, Wt1e, bt1e, KCe, cbe, Wt2e,
             bt2e, t2_ref, stats_ref):
    @pl.when(pl.program_id(0) == 0)
    def _():
        stats_ref[...] = jnp.zeros_like(stats_ref)
    scale, shift = _bn_scale_shift(sin_ref, gam, bet, float(B * 8))

    def x_t(t):
        return jnp.maximum(x_ref[t] * scale + shift, 0.0)

    h2 = _st_core(x_ref, x_t, L_ref[...], Wt1e[...], bt1e[...], KCe[...],
                  cbe[...], Wt2e[...], bt2e[...], 8)
    _write_out(h2, t2_ref, stats_ref)


def _body_out(x_ref, sin_ref, gam, bet, lw_ref, lb_ref, out_ref):
    scale, shift = _bn_scale_shift(sin_ref, gam, bet, float(B * 4))
    h = jnp.maximum(x_ref[3] * scale + shift, 0.0)
    out = lax.dot_general(h, lw_ref[...], (((1,), (1,)), ((), ())),
                          preferred_element_type=F32)
    out_ref[...] = out + lb_ref[...]


def _full(shape):
    if len(shape) == 2:
        return pl.BlockSpec(shape, lambda i: (0, 0))
    return pl.BlockSpec(shape, lambda i: (0, 0, 0))


def _prep_block(p, pref):
    eye = jnp.eye(BB, dtype=F32)
    wt1e = jnp.kron(jnp.concatenate(
        [p[pref + '_t1_w%d' % j].reshape(12, 3) for j in (1, 2, 3)],
        axis=0), eye)
    bt1e = jnp.repeat(jnp.concatenate(
        [p[pref + '_t1_b%d' % j] for j in (1, 2, 3)]), BB).reshape(36 * BB, 1)
    kce = jnp.concatenate([jnp.kron(p[pref + '_cw0'], eye),
                           jnp.kron(p[pref + '_cw1'], eye)], axis=1)
    cbe = jnp.repeat(p[pref + '_cb'], BB).reshape(12 * BB, 1)
    w2s = [p[pref + '_t2_w%d' % j].reshape(12, 3) for j in (1, 2, 3)]
    w2flat = jnp.stack([w.T.reshape(36) for w in w2s], axis=0)
    wt2e = jnp.kron(w2flat, eye)
    bt2e = jnp.repeat(jnp.stack(
        [p[pref + '_t2_b%d' % j][0] for j in (1, 2, 3)]), BB).reshape(3 * BB, 1)
    return (wt1e, bt1e, kce, cbe, wt2e, bt2e)


def _param_specs():
    return [_full((36 * BB, 3 * BB)), _full((36 * BB, 1)),
            _full((12 * BB, 24 * BB)), _full((12 * BB, 1)),
            _full((3 * BB, 36 * BB)), _full((3 * BB, 1))]


def _dense_forward(L, x3, params):
    pr1 = _prep_block(params, 'r1')
    pr2 = _prep_block(params, 'r2')
    grid = (B // BB,)

    t2a, stats_a = pl.pallas_call(
        _body_r1,
        grid=grid,
        in_specs=[pl.BlockSpec((12, BB, N), lambda i: (0, i, 0)),
                  _full((N, N))] + _param_specs(),
        out_specs=[pl.BlockSpec((8, BB, N), lambda i: (0, i, 0)),
                   _full((2, N))],
        out_shape=[jax.ShapeDtypeStruct((8, B, N), F32),
                   jax.ShapeDtypeStruct((2, N), F32)],
    )(x3, L, *pr1)

    t2b, stats_b = pl.pallas_call(
        _body_r2,
        grid=grid,
        in_specs=[pl.BlockSpec((8, BB, N), lambda i: (0, i, 0)),
                  _full((2, N)), _full((1, N)), _full((1, N)),
                  _full((N, N))] + _param_specs(),
        out_specs=[pl.BlockSpec((4, BB, N), lambda i: (0, i, 0)),
                   _full((2, N))],
        out_shape=[jax.ShapeDtypeStruct((4, B, N), F32),
                   jax.ShapeDtypeStruct((2, N), F32)],
    )(t2a, stats_a, params['r1_bn_g'].reshape(1, N),
      params['r1_bn_b'].reshape(1, N), L, *pr2)

    out = pl.pallas_call(
        _body_out,
        grid=(B // BBC,),
        in_specs=[pl.BlockSpec((4, BBC, N), lambda i: (0, i, 0)),
                  _full((2, N)), _full((1, N)), _full((1, N)),
                  _full((N, N)), _full((1, N))],
        out_specs=pl.BlockSpec((BBC, N), lambda i: (i, 0)),
        out_shape=jax.ShapeDtypeStruct((B, N), F32),
    )(t2b, stats_b, params['r2_bn_g'].reshape(1, N),
      params['r2_bn_b'].reshape(1, N), params['lin_w'],
      params['lin_b'].reshape(1, N))
    return out


def kernel(x, edge_weight, params, edge_index):
    L = _build_lhat(edge_index.astype(jnp.int32), edge_weight.astype(F32))
    x3 = jnp.transpose(x[..., 0].astype(F32), (1, 0, 2))
    return _dense_forward(L, x3, params)
